# Initial kernel scaffold; baseline (speedup 1.0000x reference)
import functools

import jax
import jax.numpy as jnp
from jax import lax
from jax.experimental import pallas as pl
from jax.experimental.pallas import tpu as pltpu
from jax.experimental.pallas import tpu_sc as plsc

_N = 10000
_D = 128
_C = 64
_E = 320000
_NC = 2
_NS = 16
_NW = _NC * _NS
_CH = 128
_CHUNKS = 80
_EPT = _CHUNKS * _CH
_EPAD = _NW * _EPT
_NPAD = 10016
_ZS = _NPAD // _NS


def _vmesh():
    return plsc.VectorSubcoreMesh(core_axis_name="c", subcore_axis_name="s")


def _sc_degree(col_t, z16, ones16):
    @functools.partial(
        pl.kernel,
        out_type=jax.ShapeDtypeStruct((_NC, _NPAD, 16), jnp.float32),
        mesh=_vmesh(),
        scratch_types=[
            pltpu.VMEM((_CHUNKS, _CH), jnp.int32),
            pltpu.VMEM((_CH, 16), jnp.float32),
            pltpu.VMEM_SHARED((_NPAD, 16), jnp.float32),
            pltpu.SemaphoreType.DMA,
        ],
    )
    def k(col_ref, z_ref, ones_ref, out_ref, col_v, ones_v, acc, sem):
        cid = lax.axis_index("c")
        sid = lax.axis_index("s")
        wid = sid * _NC + cid
        pltpu.sync_copy(z_ref, acc.at[pl.ds(sid * _ZS, _ZS)])
        pltpu.sync_copy(col_ref.at[wid], col_v)
        pltpu.sync_copy(ones_ref, ones_v)
        plsc.subcore_barrier()

        @pl.loop(0, _CHUNKS)
        def _(j):
            pltpu.sync_copy(ones_v, acc.at[col_v.at[j]], add=True)

        plsc.subcore_barrier()
        pltpu.sync_copy(acc.at[pl.ds(sid * _ZS, _ZS)],
                        out_ref.at[cid, pl.ds(sid * _ZS, _ZS)])

    return k(col_t, z16, ones16)


def _sc_hop(y, row_t, col_t, z64):
    @functools.partial(
        pl.kernel,
        out_type=jax.ShapeDtypeStruct((_NC, _NPAD, _C), jnp.float32),
        mesh=_vmesh(),
        scratch_types=[
            pltpu.VMEM((_CHUNKS, _CH), jnp.int32),
            pltpu.VMEM((_CHUNKS, _CH), jnp.int32),
            pltpu.VMEM((_CH, _C), jnp.float32),
            pltpu.VMEM_SHARED((_NPAD, _C), jnp.float32),
            pltpu.SemaphoreType.DMA,
        ],
    )
    def k(y_ref, row_ref, col_ref, z_ref, out_ref, row_v, col_v, gbuf, acc, sem):
        cid = lax.axis_index("c")
        sid = lax.axis_index("s")
        wid = sid * _NC + cid
        pltpu.sync_copy(z_ref, acc.at[pl.ds(sid * _ZS, _ZS)])
        pltpu.sync_copy(row_ref.at[wid], row_v)
        pltpu.sync_copy(col_ref.at[wid], col_v)
        plsc.subcore_barrier()

        @pl.loop(0, _CHUNKS)
        def _(j):
            pltpu.async_copy(y_ref.at[row_v.at[j]], gbuf, sem).wait()
            pltpu.sync_copy(gbuf, acc.at[col_v.at[j]], add=True)

        plsc.subcore_barrier()
        pltpu.sync_copy(acc.at[pl.ds(sid * _ZS, _ZS)],
                        out_ref.at[cid, pl.ds(sid * _ZS, _ZS)])

    return k(y, row_t, col_t, z64)


def _tc_head(x, W, degp):
    def body(x_ref, w_ref, dp_ref, y_ref, dinv_ref):
        deg = dp_ref[0, :, 0] + dp_ref[1, :, 0] + 1.0
        dinv = lax.rsqrt(deg)
        xw = jnp.dot(x_ref[...], w_ref[...], preferred_element_type=jnp.float32)
        y_ref[...] = xw * dinv[:, None]
        dinv_ref[...] = dinv[:, None]

    return pl.pallas_call(
        body,
        out_shape=(jax.ShapeDtypeStruct((_N, _C), jnp.float32),
                   jax.ShapeDtypeStruct((_N, 1), jnp.float32)),
    )(x, W, degp)


def _tc_combine_mid(sp, y, dinv):
    def body(sp_ref, y_ref, dv_ref, o_ref):
        dv = dv_ref[...]
        o_ref[...] = (sp_ref[0] + sp_ref[1] + y_ref[...]) * (dv * dv)

    return pl.pallas_call(
        body,
        out_shape=jax.ShapeDtypeStruct((_N, _C), jnp.float32),
    )(sp, y, dinv)


def _tc_combine_final(sp, y, dinv, b2):
    def body(sp_ref, y_ref, dv_ref, b_ref, o_ref):
        o_ref[...] = (sp_ref[0] + sp_ref[1] + y_ref[...]) * dv_ref[...] + b_ref[...]

    return pl.pallas_call(
        body,
        out_shape=jax.ShapeDtypeStruct((_N, _C), jnp.float32),
    )(sp, y, dinv, b2)


def kernel(x, edge_index, W, b):
    row = edge_index[0]
    col = edge_index[1]
    pad = _EPAD - _E
    rowp = jnp.concatenate([row, jnp.zeros((pad,), row.dtype)])
    colp = jnp.concatenate([col, jnp.full((pad,), _N, col.dtype)])
    row_t = rowp.reshape(_NW, _CHUNKS, _CH)
    col_t = colp.reshape(_NW, _CHUNKS, _CH)
    z64 = jnp.zeros((_ZS, _C), jnp.float32)
    z16 = jnp.zeros((_ZS, 16), jnp.float32)
    ones16 = jnp.ones((_CH, 16), jnp.float32)

    degp = _sc_degree(col_t, z16, ones16)
    y0, dinv = _tc_head(x, W, degp[:, :_N, :])
    s = _sc_hop(y0, row_t, col_t, z64)
    y1 = _tc_combine_mid(s[:, :_N], y0, dinv)
    t = _sc_hop(y1, row_t, col_t, z64)
    return _tc_combine_final(t[:, :_N], y1, dinv, b.reshape(1, _C))


# speedup vs baseline: 14.2256x; 14.2256x over previous
import functools

import jax
import jax.numpy as jnp
from jax import lax
from jax.experimental import pallas as pl
from jax.experimental.pallas import tpu as pltpu
from jax.experimental.pallas import tpu_sc as plsc

_N = 10000
_D = 128
_C = 64
_E = 320000
_NC = 2
_NS = 16
_NW = _NC * _NS
_CH = 128
_CHUNKS = 80
_EPT = _CHUNKS * _CH
_EPAD = _NW * _EPT
_NPAD = 10112
_ZS = _NPAD // _NS


def _vmesh():
    return plsc.VectorSubcoreMesh(core_axis_name="c", subcore_axis_name="s")


def _sc_degree(col_t, z16, ones16):
    @functools.partial(
        pl.kernel,
        out_type=jax.ShapeDtypeStruct((_NC, _NPAD, 16), jnp.float32),
        mesh=_vmesh(),
        compiler_params=pltpu.CompilerParams(use_tc_tiling_on_sc=False),
        scratch_types=[
            pltpu.VMEM((_CHUNKS, _CH), jnp.int32),
            pltpu.VMEM((_CH, 16), jnp.float32),
            pltpu.VMEM_SHARED((_NPAD, 16), jnp.float32),
            pltpu.SemaphoreType.DMA,
        ],
    )
    def k(col_ref, z_ref, ones_ref, out_ref, col_v, ones_v, acc, sem):
        cid = lax.axis_index("c")
        sid = lax.axis_index("s")
        wid = sid * _NC + cid
        pltpu.sync_copy(z_ref, acc.at[pl.ds(sid * _ZS, _ZS)])
        pltpu.sync_copy(col_ref.at[wid], col_v)
        pltpu.sync_copy(ones_ref, ones_v)
        plsc.subcore_barrier()

        @pl.loop(0, _CHUNKS)
        def _(j):
            pltpu.sync_copy(ones_v, acc.at[col_v.at[j]], add=True)

        plsc.subcore_barrier()
        pltpu.sync_copy(acc.at[pl.ds(sid * _ZS, _ZS)],
                        out_ref.at[cid, pl.ds(sid * _ZS, _ZS)])

    return k(col_t, z16, ones16)


def _sc_hop(y, row_t, col_t, z64):
    @functools.partial(
        pl.kernel,
        out_type=jax.ShapeDtypeStruct((_NC, _NPAD, _C), jnp.float32),
        mesh=_vmesh(),
        compiler_params=pltpu.CompilerParams(use_tc_tiling_on_sc=False),
        scratch_types=[
            pltpu.VMEM((_CHUNKS, _CH), jnp.int32),
            pltpu.VMEM((_CHUNKS, _CH), jnp.int32),
            pltpu.VMEM((_CH, _C), jnp.float32),
            pltpu.VMEM_SHARED((_NPAD, _C), jnp.float32),
            pltpu.SemaphoreType.DMA,
        ],
    )
    def k(y_ref, row_ref, col_ref, z_ref, out_ref, row_v, col_v, gbuf, acc, sem):
        cid = lax.axis_index("c")
        sid = lax.axis_index("s")
        wid = sid * _NC + cid
        pltpu.sync_copy(z_ref, acc.at[pl.ds(sid * _ZS, _ZS)])
        pltpu.sync_copy(row_ref.at[wid], row_v)
        pltpu.sync_copy(col_ref.at[wid], col_v)
        plsc.subcore_barrier()

        @pl.loop(0, _CHUNKS)
        def _(j):
            pltpu.async_copy(y_ref.at[row_v.at[j]], gbuf, sem).wait()
            pltpu.sync_copy(gbuf, acc.at[col_v.at[j]], add=True)

        plsc.subcore_barrier()
        pltpu.sync_copy(acc.at[pl.ds(sid * _ZS, _ZS)],
                        out_ref.at[cid, pl.ds(sid * _ZS, _ZS)])

    return k(y, row_t, col_t, z64)


def _tc_head(x, W, degp):
    def body(x_ref, w_ref, dp_ref, y_ref, dinv_ref):
        deg = dp_ref[0, :, 0] + dp_ref[1, :, 0] + 1.0
        dinv = lax.rsqrt(deg)
        xw = jnp.dot(x_ref[...], w_ref[...], preferred_element_type=jnp.float32)
        y_ref[...] = xw * dinv[:, None]
        dinv_ref[...] = dinv[:, None]

    return pl.pallas_call(
        body,
        out_shape=(jax.ShapeDtypeStruct((_N, _C), jnp.float32),
                   jax.ShapeDtypeStruct((_N, 1), jnp.float32)),
    )(x, W, degp)


def _tc_combine_mid(sp, y, dinv):
    def body(sp_ref, y_ref, dv_ref, o_ref):
        dv = dv_ref[...]
        o_ref[...] = (sp_ref[0] + sp_ref[1] + y_ref[...]) * (dv * dv)

    return pl.pallas_call(
        body,
        out_shape=jax.ShapeDtypeStruct((_N, _C), jnp.float32),
    )(sp, y, dinv)


def _tc_combine_final(sp, y, dinv, b2):
    def body(sp_ref, y_ref, dv_ref, b_ref, o_ref):
        o_ref[...] = (sp_ref[0] + sp_ref[1] + y_ref[...]) * dv_ref[...] + b_ref[...]

    return pl.pallas_call(
        body,
        out_shape=jax.ShapeDtypeStruct((_N, _C), jnp.float32),
    )(sp, y, dinv, b2)


def kernel(x, edge_index, W, b):
    row = edge_index[0]
    col = edge_index[1]
    pad = _EPAD - _E
    rowp = jnp.concatenate([row, jnp.zeros((pad,), row.dtype)])
    colp = jnp.concatenate([col, jnp.full((pad,), _N, col.dtype)])
    row_t = rowp.reshape(_NW, _CHUNKS, _CH)
    col_t = colp.reshape(_NW, _CHUNKS, _CH)
    z64 = jnp.zeros((_ZS, _C), jnp.float32)
    z16 = jnp.zeros((_ZS, 16), jnp.float32)
    ones16 = jnp.ones((_CH, 16), jnp.float32)

    degp = _sc_degree(col_t, z16, ones16)
    y0, dinv = _tc_head(x, W, degp[:, :_N, :])
    s = _sc_hop(y0, row_t, col_t, z64)
    y1 = _tc_combine_mid(s[:, :_N], y0, dinv)
    t = _sc_hop(y1, row_t, col_t, z64)
    return _tc_combine_final(t[:, :_N], y1, dinv, b.reshape(1, _C))


# speedup vs baseline: 15.7028x; 1.1038x over previous
import functools

import jax
import jax.numpy as jnp
from jax import lax
from jax.experimental import pallas as pl
from jax.experimental.pallas import tpu as pltpu
from jax.experimental.pallas import tpu_sc as plsc

_N = 10000
_D = 128
_C = 64
_E = 320000
_NC = 2
_NS = 16
_NW = _NC * _NS
_CH = 128
_CHUNKS = 80
_EPT = _CHUNKS * _CH
_EPAD = _NW * _EPT
_NPAD = 10112
_ZS = _NPAD // _NS
_NB = 8
_PD = 4


def _vmesh():
    return plsc.VectorSubcoreMesh(core_axis_name="c", subcore_axis_name="s")


def _sc_degree(col_t, z16, ones16):
    @functools.partial(
        pl.kernel,
        out_type=jax.ShapeDtypeStruct((_NC, _NPAD, 16), jnp.float32),
        mesh=_vmesh(),
        compiler_params=pltpu.CompilerParams(use_tc_tiling_on_sc=False),
        scratch_types=[
            pltpu.VMEM((_CHUNKS, _CH), jnp.int32),
            pltpu.VMEM((_CH, 16), jnp.float32),
            pltpu.VMEM_SHARED((_NPAD, 16), jnp.float32),
            pltpu.SemaphoreType.DMA,
        ],
    )
    def k(col_ref, z_ref, ones_ref, out_ref, col_v, ones_v, acc, sem):
        cid = lax.axis_index("c")
        sid = lax.axis_index("s")
        wid = sid * _NC + cid
        pltpu.sync_copy(z_ref, acc.at[pl.ds(sid * _ZS, _ZS)])
        pltpu.sync_copy(col_ref.at[wid], col_v)
        pltpu.sync_copy(ones_ref, ones_v)
        plsc.subcore_barrier()

        @pl.loop(0, _CHUNKS)
        def _(j):
            pltpu.sync_copy(ones_v, acc.at[col_v.at[j]], add=True)

        plsc.subcore_barrier()
        pltpu.sync_copy(acc.at[pl.ds(sid * _ZS, _ZS)],
                        out_ref.at[cid, pl.ds(sid * _ZS, _ZS)])

    return k(col_t, z16, ones16)


def _sc_hop(y, row_t, col_t, z64):
    @functools.partial(
        pl.kernel,
        out_type=jax.ShapeDtypeStruct((_NC, _NPAD, _C), jnp.float32),
        mesh=_vmesh(),
        compiler_params=pltpu.CompilerParams(use_tc_tiling_on_sc=False),
        scratch_types=[
            pltpu.VMEM((_CHUNKS, _CH), jnp.int32),
            pltpu.VMEM((_CHUNKS, _CH), jnp.int32),
            [pltpu.VMEM((_CH, _C), jnp.float32) for _ in range(_NB)],
            pltpu.SemaphoreType.DMA((_NB,)),
            pltpu.SemaphoreType.DMA((_NB,)),
            pltpu.VMEM_SHARED((_NPAD, _C), jnp.float32),
        ],
    )
    def k(y_ref, row_ref, col_ref, z_ref, out_ref, row_v, col_v, gb, semg,
          sems, acc):
        cid = lax.axis_index("c")
        sid = lax.axis_index("s")
        wid = sid * _NC + cid
        pltpu.sync_copy(z_ref, acc.at[pl.ds(sid * _ZS, _ZS)])
        pltpu.sync_copy(row_ref.at[wid], row_v)
        pltpu.sync_copy(col_ref.at[wid], col_v)
        plsc.subcore_barrier()

        def gather(c, p):
            pltpu.async_copy(y_ref.at[row_v.at[c]], gb[p], semg.at[p])

        def wait_gather(c, p):
            pltpu.make_async_copy(y_ref.at[row_v.at[c]], gb[p],
                                  semg.at[p]).wait()

        def scatter(c, p):
            pltpu.async_copy(gb[p], acc.at[col_v.at[c]], sems.at[p], add=True)

        def wait_scatter(c, p):
            pltpu.make_async_copy(gb[p], acc.at[col_v.at[c]],
                                  sems.at[p]).wait()

        for p in range(_PD):
            gather(p, p)

        @pl.loop(0, _CHUNKS // _NB)
        def _(t):
            j0 = t * _NB
            for p in range(_NB):
                j = j0 + p
                pn = (p + _PD) % _NB

                @pl.when(j + _PD >= _NB)
                def _():
                    wait_scatter(j + _PD - _NB, pn)

                @pl.when(j + _PD < _CHUNKS)
                def _():
                    gather(j + _PD, pn)

                wait_gather(j, p)
                scatter(j, p)

        for c in range(_CHUNKS - _NB, _CHUNKS):
            if c + _NB - _PD > _CHUNKS - 1:
                wait_scatter(c, c % _NB)

        plsc.subcore_barrier()
        pltpu.sync_copy(acc.at[pl.ds(sid * _ZS, _ZS)],
                        out_ref.at[cid, pl.ds(sid * _ZS, _ZS)])

    return k(y, row_t, col_t, z64)


def _tc_head(x, W, degp):
    def body(x_ref, w_ref, dp_ref, y_ref, dinv_ref):
        deg = dp_ref[0, :, 0] + dp_ref[1, :, 0] + 1.0
        dinv = lax.rsqrt(deg)
        xw = jnp.dot(x_ref[...], w_ref[...], preferred_element_type=jnp.float32)
        y_ref[...] = xw * dinv[:, None]
        dinv_ref[...] = dinv[:, None]

    return pl.pallas_call(
        body,
        out_shape=(jax.ShapeDtypeStruct((_N, _C), jnp.float32),
                   jax.ShapeDtypeStruct((_N, 1), jnp.float32)),
    )(x, W, degp)


def _tc_combine_mid(sp, y, dinv):
    def body(sp_ref, y_ref, dv_ref, o_ref):
        dv = dv_ref[...]
        o_ref[...] = (sp_ref[0] + sp_ref[1] + y_ref[...]) * (dv * dv)

    return pl.pallas_call(
        body,
        out_shape=jax.ShapeDtypeStruct((_N, _C), jnp.float32),
    )(sp, y, dinv)


def _tc_combine_final(sp, y, dinv, b2):
    def body(sp_ref, y_ref, dv_ref, b_ref, o_ref):
        o_ref[...] = (sp_ref[0] + sp_ref[1] + y_ref[...]) * dv_ref[...] + b_ref[...]

    return pl.pallas_call(
        body,
        out_shape=jax.ShapeDtypeStruct((_N, _C), jnp.float32),
    )(sp, y, dinv, b2)


def kernel(x, edge_index, W, b):
    row = edge_index[0]
    col = edge_index[1]
    pad = _EPAD - _E
    rowp = jnp.concatenate([row, jnp.zeros((pad,), row.dtype)])
    colp = jnp.concatenate([col, jnp.full((pad,), _N, col.dtype)])
    row_t = rowp.reshape(_NW, _CHUNKS, _CH)
    col_t = colp.reshape(_NW, _CHUNKS, _CH)
    z64 = jnp.zeros((_ZS, _C), jnp.float32)
    z16 = jnp.zeros((_ZS, 16), jnp.float32)
    ones16 = jnp.ones((_CH, 16), jnp.float32)

    degp = _sc_degree(col_t, z16, ones16)
    y0, dinv = _tc_head(x, W, degp[:, :_N, :])
    s = _sc_hop(y0, row_t, col_t, z64)
    y1 = _tc_combine_mid(s[:, :_N], y0, dinv)
    t = _sc_hop(y1, row_t, col_t, z64)
    return _tc_combine_final(t[:, :_N], y1, dinv, b.reshape(1, _C))


# speedup vs baseline: 29.5189x; 1.8798x over previous
import functools

import jax
import jax.numpy as jnp
from jax import lax
from jax.experimental import pallas as pl
from jax.experimental.pallas import tpu as pltpu
from jax.experimental.pallas import tpu_sc as plsc

_N = 10000
_D = 128
_C = 64
_HC = _C // 2
_E = 320000
_NC = 2
_NS = 16
_NW = _NC * _NS
_CH = 128
_CHUNKS = 80
_EPT = _CHUNKS * _CH
_EPAD = _NW * _EPT
_NPAD = 10112
_ZS = _NPAD // _NS
_NB = 8
_PD = 4
_YS = _N // _NS


def _vmesh():
    return plsc.VectorSubcoreMesh(core_axis_name="c", subcore_axis_name="s")


def _sc_degree(col_t, z16, ones16):
    @functools.partial(
        pl.kernel,
        out_type=jax.ShapeDtypeStruct((_NC, _NPAD, 16), jnp.float32),
        mesh=_vmesh(),
        compiler_params=pltpu.CompilerParams(use_tc_tiling_on_sc=False),
        scratch_types=[
            pltpu.VMEM((_CHUNKS, _CH), jnp.int32),
            pltpu.VMEM((_CH, 16), jnp.float32),
            pltpu.VMEM_SHARED((_NPAD, 16), jnp.float32),
            pltpu.SemaphoreType.DMA,
        ],
    )
    def k(col_ref, z_ref, ones_ref, out_ref, col_v, ones_v, acc, sem):
        cid = lax.axis_index("c")
        sid = lax.axis_index("s")
        wid = sid * _NC + cid
        pltpu.sync_copy(z_ref, acc.at[pl.ds(sid * _ZS, _ZS)])
        pltpu.sync_copy(col_ref.at[wid], col_v)
        pltpu.sync_copy(ones_ref, ones_v)
        plsc.subcore_barrier()

        @pl.loop(0, _CHUNKS)
        def _(j):
            pltpu.sync_copy(ones_v, acc.at[col_v.at[j]], add=True)

        plsc.subcore_barrier()
        pltpu.sync_copy(acc.at[pl.ds(sid * _ZS, _ZS)],
                        out_ref.at[cid, pl.ds(sid * _ZS, _ZS)])

    return k(col_t, z16, ones16)


def _sc_hop(ya, yb, row_t, col_t, z32):
    out_t = jax.ShapeDtypeStruct((_NC, _NPAD, _HC), jnp.float32)

    @functools.partial(
        pl.kernel,
        out_type=(out_t, out_t),
        mesh=_vmesh(),
        compiler_params=pltpu.CompilerParams(use_tc_tiling_on_sc=False),
        scratch_types=[
            pltpu.VMEM((_CHUNKS, _CH), jnp.int32),
            pltpu.VMEM((_CHUNKS, _CH), jnp.int32),
            [pltpu.VMEM((_CH, _HC), jnp.float32) for _ in range(_NB)],
            pltpu.SemaphoreType.DMA((_NB,)),
            pltpu.SemaphoreType.DMA((_NB,)),
            pltpu.VMEM_SHARED((_NPAD, _HC), jnp.float32),
            pltpu.VMEM_SHARED((_N, _HC), jnp.float32),
        ],
    )
    def k(ya_ref, yb_ref, row_ref, col_ref, z_ref, oa_ref, ob_ref,
          row_v, col_v, gb, semg, sems, acc, ysh):
        cid = lax.axis_index("c")
        sid = lax.axis_index("s")
        wid = sid * _NC + cid
        pltpu.sync_copy(row_ref.at[wid], row_v)
        pltpu.sync_copy(col_ref.at[wid], col_v)

        def gather(c, p):
            pltpu.async_copy(ysh.at[row_v.at[c]], gb[p], semg.at[p])

        def wait_gather(c, p):
            pltpu.make_async_copy(ysh.at[row_v.at[c]], gb[p],
                                  semg.at[p]).wait()

        def scatter(c, p):
            pltpu.async_copy(gb[p], acc.at[col_v.at[c]], sems.at[p], add=True)

        def wait_scatter(c, p):
            pltpu.make_async_copy(gb[p], acc.at[col_v.at[c]],
                                  sems.at[p]).wait()

        for h, (y_ref, o_ref) in enumerate(((ya_ref, oa_ref),
                                            (yb_ref, ob_ref))):
            if h:
                plsc.subcore_barrier()
            pltpu.sync_copy(y_ref.at[pl.ds(sid * _YS, _YS)],
                            ysh.at[pl.ds(sid * _YS, _YS)])
            pltpu.sync_copy(z_ref, acc.at[pl.ds(sid * _ZS, _ZS)])
            plsc.subcore_barrier()

            for p in range(_PD):
                gather(p, p)

            @pl.loop(0, _CHUNKS // _NB)
            def _(t):
                j0 = t * _NB
                for p in range(_NB):
                    j = j0 + p
                    pn = (p + _PD) % _NB

                    @pl.when(j + _PD >= _NB)
                    def _():
                        wait_scatter(j + _PD - _NB, pn)

                    @pl.when(j + _PD < _CHUNKS)
                    def _():
                        gather(j + _PD, pn)

                    wait_gather(j, p)
                    scatter(j, p)

            for c in range(_CHUNKS - _NB, _CHUNKS):
                if c + _NB - _PD > _CHUNKS - 1:
                    wait_scatter(c, c % _NB)

            plsc.subcore_barrier()
            pltpu.sync_copy(acc.at[pl.ds(sid * _ZS, _ZS)],
                            o_ref.at[cid, pl.ds(sid * _ZS, _ZS)])

    return k(ya, yb, row_t, col_t, z32)


def _tc_head(x, W, degp):
    def body(x_ref, w_ref, dp_ref, ya_ref, yb_ref, dinv_ref):
        deg = dp_ref[0, :, 0] + dp_ref[1, :, 0] + 1.0
        dinv = lax.rsqrt(deg)
        xw = jnp.dot(x_ref[...], w_ref[...], preferred_element_type=jnp.float32)
        y = xw * dinv[:, None]
        ya_ref[...] = y[:, :_HC]
        yb_ref[...] = y[:, _HC:]
        dinv_ref[...] = dinv[:, None]

    return pl.pallas_call(
        body,
        out_shape=(jax.ShapeDtypeStruct((_N, _HC), jnp.float32),
                   jax.ShapeDtypeStruct((_N, _HC), jnp.float32),
                   jax.ShapeDtypeStruct((_N, 1), jnp.float32)),
    )(x, W, degp)


def _tc_combine_mid(sa, sb, ya, yb, dinv):
    def body(sa_ref, sb_ref, ya_ref, yb_ref, dv_ref, oa_ref, ob_ref):
        dv2 = dv_ref[...] * dv_ref[...]
        oa_ref[...] = (sa_ref[0] + sa_ref[1] + ya_ref[...]) * dv2
        ob_ref[...] = (sb_ref[0] + sb_ref[1] + yb_ref[...]) * dv2

    return pl.pallas_call(
        body,
        out_shape=(jax.ShapeDtypeStruct((_N, _HC), jnp.float32),
                   jax.ShapeDtypeStruct((_N, _HC), jnp.float32)),
    )(sa, sb, ya, yb, dinv)


def _tc_combine_final(ta, tb, ya, yb, dinv, b2):
    def body(ta_ref, tb_ref, ya_ref, yb_ref, dv_ref, b_ref, o_ref):
        dv = dv_ref[...]
        ha = (ta_ref[0] + ta_ref[1] + ya_ref[...]) * dv
        hb = (tb_ref[0] + tb_ref[1] + yb_ref[...]) * dv
        o_ref[...] = jnp.concatenate([ha, hb], axis=1) + b_ref[...]

    return pl.pallas_call(
        body,
        out_shape=jax.ShapeDtypeStruct((_N, _C), jnp.float32),
    )(ta, tb, ya, yb, dinv, b2)


def kernel(x, edge_index, W, b):
    row = edge_index[0]
    col = edge_index[1]
    pad = _EPAD - _E
    rowp = jnp.concatenate([row, jnp.zeros((pad,), row.dtype)])
    colp = jnp.concatenate([col, jnp.full((pad,), _N, col.dtype)])
    row_t = rowp.reshape(_NW, _CHUNKS, _CH)
    col_t = colp.reshape(_NW, _CHUNKS, _CH)
    z32 = jnp.zeros((_ZS, _HC), jnp.float32)
    z16 = jnp.zeros((_ZS, 16), jnp.float32)
    ones16 = jnp.ones((_CH, 16), jnp.float32)

    degp = _sc_degree(col_t, z16, ones16)
    ya0, yb0, dinv = _tc_head(x, W, degp[:, :_N, :])
    sa, sb = _sc_hop(ya0, yb0, row_t, col_t, z32)
    ya1, yb1 = _tc_combine_mid(sa[:, :_N], sb[:, :_N], ya0, yb0, dinv)
    ta, tb = _sc_hop(ya1, yb1, row_t, col_t, z32)
    return _tc_combine_final(ta[:, :_N], tb[:, :_N], ya1, yb1, dinv,
                             b.reshape(1, _C))


# speedup vs baseline: 40.0565x; 1.3570x over previous
import functools

import jax
import jax.numpy as jnp
from jax import lax
from jax.experimental import pallas as pl
from jax.experimental.pallas import tpu as pltpu
from jax.experimental.pallas import tpu_sc as plsc

_N = 10000
_D = 128
_C = 64
_HC = _C // 2
_E = 320000
_NC = 2
_NS = 16
_NW = _NC * _NS
_CH = 128
_CHD = 80
_CHH = 160
_EPAD = _NW * _CHD * _CH
_NPAD = 10112
_ZS = _NPAD // _NS
_YS = _N // _NS
_NB = 4
_PD = 2
_LASTY = 15 * _ZS
_LASTN = _N - _LASTY


def _vmesh():
    return plsc.VectorSubcoreMesh(core_axis_name="c", subcore_axis_name="s")


def _sc_degree(col_t, z16, ones16):
    @functools.partial(
        pl.kernel,
        out_type=jax.ShapeDtypeStruct((_NC, _NPAD, 16), jnp.float32),
        mesh=_vmesh(),
        compiler_params=pltpu.CompilerParams(use_tc_tiling_on_sc=False),
        scratch_types=[
            pltpu.VMEM((_CHD, _CH), jnp.int32),
            pltpu.VMEM((_CH, 16), jnp.float32),
            pltpu.VMEM_SHARED((_NPAD, 16), jnp.float32),
            pltpu.SemaphoreType.DMA,
        ],
    )
    def k(col_ref, z_ref, ones_ref, out_ref, col_v, ones_v, acc, sem):
        cid = lax.axis_index("c")
        sid = lax.axis_index("s")
        wid = sid * _NC + cid
        pltpu.sync_copy(z_ref, acc.at[pl.ds(sid * _ZS, _ZS)])
        pltpu.sync_copy(col_ref.at[wid], col_v)
        pltpu.sync_copy(ones_ref, ones_v)
        plsc.subcore_barrier()

        @pl.loop(0, _CHD)
        def _(j):
            pltpu.sync_copy(ones_v, acc.at[col_v.at[j]], add=True)

        plsc.subcore_barrier()
        pltpu.sync_copy(acc.at[pl.ds(sid * _ZS, _ZS)],
                        out_ref.at[cid, pl.ds(sid * _ZS, _ZS)])

    return k(col_t, z16, ones16)


def _sc_hops(ya, yb, dinv2_16, row_t, col_t, zpad):
    @functools.partial(
        pl.kernel,
        out_type=jax.ShapeDtypeStruct((_NC, _NPAD, _HC), jnp.float32),
        mesh=_vmesh(),
        compiler_params=pltpu.CompilerParams(use_tc_tiling_on_sc=False),
        scratch_types=[
            pltpu.VMEM((_CHH, _CH), jnp.int32),
            pltpu.VMEM((_CHH, _CH), jnp.int32),
            [pltpu.VMEM((_CH, _HC), jnp.float32) for _ in range(_NB)],
            pltpu.VMEM((_ZS, _HC), jnp.float32),
            pltpu.VMEM((_ZS, 16), jnp.float32),
            pltpu.SemaphoreType.DMA((_NB,)),
            pltpu.SemaphoreType.DMA((_NB,)),
            pltpu.VMEM_SHARED((_NPAD, _HC), jnp.float32),
            pltpu.VMEM_SHARED((_N, _HC), jnp.float32),
        ],
    )
    def k(ya_ref, yb_ref, d_ref, row_ref, col_ref, zpad_ref, o_ref,
          row_v, col_v, gb, ybuf, dv, semg, sems, acc, ysh):
        cid = lax.axis_index("c")
        sid = lax.axis_index("s")
        pltpu.sync_copy(row_ref.at[sid], row_v)
        pltpu.sync_copy(col_ref.at[sid], col_v)
        pltpu.sync_copy(d_ref.at[pl.ds(sid * _ZS, _ZS)], dv)

        def stage(y_ref):
            pltpu.sync_copy(y_ref.at[pl.ds(sid * _YS, _YS)],
                            ysh.at[pl.ds(sid * _YS, _YS)])

            @pl.when(sid < _NS - 1)
            def _():
                pltpu.sync_copy(y_ref.at[pl.ds(sid * _ZS, _ZS)],
                                acc.at[pl.ds(sid * _ZS, _ZS)])

            @pl.when(sid == _NS - 1)
            def _():
                pltpu.sync_copy(y_ref.at[pl.ds(_LASTY, _LASTN)],
                                acc.at[pl.ds(_LASTY, _LASTN)])
                pltpu.sync_copy(zpad_ref, acc.at[pl.ds(_N, _NPAD - _N)])

        @pl.when(cid == 0)
        def _():
            stage(ya_ref)

        @pl.when(cid == 1)
        def _():
            stage(yb_ref)

        plsc.subcore_barrier()

        def gather(c, p):
            pltpu.async_copy(ysh.at[row_v.at[c]], gb[p], semg.at[p])

        def wait_gather(c, p):
            pltpu.make_async_copy(ysh.at[row_v.at[c]], gb[p],
                                  semg.at[p]).wait()

        def scatter(c, p):
            pltpu.async_copy(gb[p], acc.at[col_v.at[c]], sems.at[p], add=True)

        def wait_scatter(c, p):
            pltpu.make_async_copy(gb[p], acc.at[col_v.at[c]],
                                  sems.at[p]).wait()

        def hop_loop():
            for p in range(_PD):
                gather(p, p)

            @pl.loop(0, _CHH // _NB)
            def _(t):
                j0 = t * _NB
                for p in range(_NB):
                    j = j0 + p
                    pn = (p + _PD) % _NB

                    @pl.when(j + _PD >= _NB)
                    def _():
                        wait_scatter(j + _PD - _NB, pn)

                    @pl.when(j + _PD < _CHH)
                    def _():
                        gather(j + _PD, pn)

                    wait_gather(j, p)
                    scatter(j, p)

            for c in range(_CHH - _NB, _CHH):
                if c + _NB - _PD > _CHH - 1:
                    wait_scatter(c, c % _NB)

            plsc.subcore_barrier()

        hop_loop()

        pltpu.sync_copy(acc.at[pl.ds(sid * _ZS, _ZS)], ybuf)

        @pl.loop(0, _ZS)
        def _(r):
            s = dv[r]
            for q in (0, 16):
                ybuf[r, pl.ds(q, 16)] = ybuf[r, pl.ds(q, 16)] * s

        pltpu.sync_copy(ybuf, acc.at[pl.ds(sid * _ZS, _ZS)])

        @pl.when(sid < _NS - 1)
        def _():
            pltpu.sync_copy(ybuf, ysh.at[pl.ds(sid * _ZS, _ZS)])

        @pl.when(sid == _NS - 1)
        def _():
            pltpu.sync_copy(ybuf.at[pl.ds(0, _LASTN)],
                            ysh.at[pl.ds(_LASTY, _LASTN)])

        plsc.subcore_barrier()

        hop_loop()

        pltpu.sync_copy(acc.at[pl.ds(sid * _ZS, _ZS)],
                        o_ref.at[cid, pl.ds(sid * _ZS, _ZS)])

    return k(ya, yb, dinv2_16, row_t, col_t, zpad)


def _tc_head(x, W, degp):
    def body(x_ref, w_ref, dp_ref, ya_ref, yb_ref, dinv_ref):
        deg = dp_ref[0, :, 0] + dp_ref[1, :, 0] + 1.0
        dinv = lax.rsqrt(deg)
        xw = jnp.dot(x_ref[...], w_ref[...], preferred_element_type=jnp.float32)
        y = xw * dinv[:_N, None]
        ya_ref[...] = y[:, :_HC]
        yb_ref[...] = y[:, _HC:]
        dinv_ref[...] = dinv[:, None]

    return pl.pallas_call(
        body,
        out_shape=(jax.ShapeDtypeStruct((_N, _HC), jnp.float32),
                   jax.ShapeDtypeStruct((_N, _HC), jnp.float32),
                   jax.ShapeDtypeStruct((_NPAD, 1), jnp.float32)),
    )(x, W, degp)


def _tc_final(t, dinv, b2):
    def body(t_ref, dv_ref, b_ref, o_ref):
        dv = dv_ref[...]
        o_ref[...] = (jnp.concatenate([t_ref[0] * dv, t_ref[1] * dv], axis=1)
                      + b_ref[...])

    return pl.pallas_call(
        body,
        out_shape=jax.ShapeDtypeStruct((_N, _C), jnp.float32),
    )(t, dinv, b2)


def kernel(x, edge_index, W, b):
    row = edge_index[0]
    col = edge_index[1]
    pad = _EPAD - _E
    rowp = jnp.concatenate([row, jnp.zeros((pad,), row.dtype)])
    colp = jnp.concatenate([col, jnp.full((pad,), _N, col.dtype)])
    col_t32 = colp.reshape(_NW, _CHD, _CH)
    row_t16 = rowp.reshape(_NS, _CHH, _CH)
    col_t16 = colp.reshape(_NS, _CHH, _CH)
    z16 = jnp.zeros((_ZS, 16), jnp.float32)
    ones16 = jnp.ones((_CH, 16), jnp.float32)
    zpad = jnp.zeros((_NPAD - _N, _HC), jnp.float32)

    degp = _sc_degree(col_t32, z16, ones16)
    ya0, yb0, dinv = _tc_head(x, W, degp)
    d2 = jnp.broadcast_to(dinv * dinv, (_NPAD, 16))
    t = _sc_hops(ya0, yb0, d2, row_t16, col_t16, zpad)
    return _tc_final(t[:, :_N], dinv[:_N], b.reshape(1, _C))


# speedup vs baseline: 42.9841x; 1.0731x over previous
import functools

import jax
import jax.numpy as jnp
from jax import lax
from jax.experimental import pallas as pl
from jax.experimental.pallas import tpu as pltpu
from jax.experimental.pallas import tpu_sc as plsc

_N = 10000
_D = 128
_C = 64
_HC = _C // 2
_E = 320000
_NC = 2
_NS = 16
_CH = 128
_CHH = 160
_EPAD = _NS * _CHH * _CH
_NPAD = 10112
_ZS = _NPAD // _NS
_HP = _NPAD // _NC
_HS = _HP // _NS
_NB = 4
_PD = 2
_LASTY = 15 * _ZS
_LASTN = _N - _LASTY


def _vmesh():
    return plsc.VectorSubcoreMesh(core_axis_name="c", subcore_axis_name="s")


def _sc_degree_dinv(col_t, z16, ones16):
    @functools.partial(
        pl.kernel,
        out_type=jax.ShapeDtypeStruct((_NPAD, 16), jnp.float32),
        mesh=_vmesh(),
        compiler_params=pltpu.CompilerParams(use_tc_tiling_on_sc=False),
        scratch_types=[
            pltpu.VMEM((_CHH, _CH), jnp.int32),
            pltpu.VMEM((_CH, 16), jnp.float32),
            pltpu.VMEM((_HS, 16), jnp.float32),
            pltpu.VMEM_SHARED((_NPAD, 16), jnp.float32),
        ],
    )
    def k(col_ref, z_ref, ones_ref, out_ref, col_v, ones_v, dbuf, acc):
        cid = lax.axis_index("c")
        sid = lax.axis_index("s")
        pltpu.sync_copy(z_ref, acc.at[pl.ds(sid * _ZS, _ZS)])
        pltpu.sync_copy(col_ref.at[sid], col_v)
        pltpu.sync_copy(ones_ref, ones_v)
        plsc.subcore_barrier()

        @pl.loop(0, _CHH)
        def _(j):
            pltpu.sync_copy(ones_v, acc.at[col_v.at[j]], add=True)

        plsc.subcore_barrier()
        off = cid * _HP + sid * _HS
        pltpu.sync_copy(acc.at[pl.ds(off, _HS)], dbuf)

        @pl.loop(0, _HS)
        def _(r):
            v = dbuf[r] + 1.0
            i = lax.bitcast_convert_type(v, jnp.int32)
            i = 0x5F3759DF - lax.shift_right_logical(i, 1)
            g = lax.bitcast_convert_type(i, jnp.float32)
            h = v * 0.5
            for _ in range(3):
                g = g * (1.5 - h * g * g)
            dbuf[r] = g

        pltpu.sync_copy(dbuf, out_ref.at[pl.ds(off, _HS)])

    return k(col_t, z16, ones16)


def _sc_hops(xwa, xwb, dinv16, row_t, col_t, zpad):
    @functools.partial(
        pl.kernel,
        out_type=jax.ShapeDtypeStruct((_NC, _NPAD, _HC), jnp.float32),
        mesh=_vmesh(),
        compiler_params=pltpu.CompilerParams(use_tc_tiling_on_sc=False),
        scratch_types=[
            pltpu.VMEM((_CHH, _CH), jnp.int32),
            pltpu.VMEM((_CHH, _CH), jnp.int32),
            [pltpu.VMEM((_CH, _HC), jnp.float32) for _ in range(_NB)],
            pltpu.VMEM((_ZS, _HC), jnp.float32),
            pltpu.VMEM((_ZS, 16), jnp.float32),
            pltpu.SemaphoreType.DMA((_NB,)),
            pltpu.SemaphoreType.DMA((_NB,)),
            pltpu.VMEM_SHARED((_NPAD, _HC), jnp.float32),
            pltpu.VMEM_SHARED((_N, _HC), jnp.float32),
        ],
    )
    def k(xwa_ref, xwb_ref, d_ref, row_ref, col_ref, zpad_ref, o_ref,
          row_v, col_v, gb, ybuf, dv, semg, sems, acc, ysh):
        cid = lax.axis_index("c")
        sid = lax.axis_index("s")
        pltpu.sync_copy(row_ref.at[sid], row_v)
        pltpu.sync_copy(col_ref.at[sid], col_v)
        pltpu.sync_copy(d_ref.at[pl.ds(sid * _ZS, _ZS)], dv)

        def scale_rows(n):
            @pl.loop(0, n)
            def _(r):
                s = dv[r]
                for q in (0, 16):
                    ybuf[r, pl.ds(q, 16)] = ybuf[r, pl.ds(q, 16)] * s

        def stage(xw_ref):
            @pl.when(sid < _NS - 1)
            def _():
                sl = pl.ds(sid * _ZS, _ZS)
                pltpu.sync_copy(xw_ref.at[sl], ybuf)
                scale_rows(_ZS)
                pltpu.sync_copy(ybuf, ysh.at[sl])
                pltpu.sync_copy(ybuf, acc.at[sl])

            @pl.when(sid == _NS - 1)
            def _():
                sl = pl.ds(_LASTY, _LASTN)
                bl = pl.ds(0, _LASTN)
                pltpu.sync_copy(xw_ref.at[sl], ybuf.at[bl])
                scale_rows(_LASTN)
                pltpu.sync_copy(ybuf.at[bl], ysh.at[sl])
                pltpu.sync_copy(ybuf.at[bl], acc.at[sl])
                pltpu.sync_copy(zpad_ref, acc.at[pl.ds(_N, _NPAD - _N)])

        @pl.when(cid == 0)
        def _():
            stage(xwa_ref)

        @pl.when(cid == 1)
        def _():
            stage(xwb_ref)

        plsc.subcore_barrier()

        def gather(c, p):
            pltpu.async_copy(ysh.at[row_v.at[c]], gb[p], semg.at[p])

        def wait_gather(c, p):
            pltpu.make_async_copy(ysh.at[row_v.at[c]], gb[p],
                                  semg.at[p]).wait()

        def scatter(c, p):
            pltpu.async_copy(gb[p], acc.at[col_v.at[c]], sems.at[p], add=True)

        def wait_scatter(c, p):
            pltpu.make_async_copy(gb[p], acc.at[col_v.at[c]],
                                  sems.at[p]).wait()

        def hop_loop():
            for p in range(_PD):
                gather(p, p)

            @pl.loop(0, _CHH // _NB)
            def _(t):
                j0 = t * _NB
                for p in range(_NB):
                    j = j0 + p
                    pn = (p + _PD) % _NB

                    @pl.when(j + _PD >= _NB)
                    def _():
                        wait_scatter(j + _PD - _NB, pn)

                    @pl.when(j + _PD < _CHH)
                    def _():
                        gather(j + _PD, pn)

                    wait_gather(j, p)
                    scatter(j, p)

            for c in range(_CHH - _NB, _CHH):
                if c + _NB - _PD > _CHH - 1:
                    wait_scatter(c, c % _NB)

            plsc.subcore_barrier()

        hop_loop()

        pltpu.sync_copy(acc.at[pl.ds(sid * _ZS, _ZS)], ybuf)

        @pl.loop(0, _ZS)
        def _(r):
            s = dv[r]
            s2 = s * s
            for q in (0, 16):
                ybuf[r, pl.ds(q, 16)] = ybuf[r, pl.ds(q, 16)] * s2

        pltpu.sync_copy(ybuf, acc.at[pl.ds(sid * _ZS, _ZS)])

        @pl.when(sid < _NS - 1)
        def _():
            pltpu.sync_copy(ybuf, ysh.at[pl.ds(sid * _ZS, _ZS)])

        @pl.when(sid == _NS - 1)
        def _():
            pltpu.sync_copy(ybuf.at[pl.ds(0, _LASTN)],
                            ysh.at[pl.ds(_LASTY, _LASTN)])

        plsc.subcore_barrier()

        hop_loop()

        pltpu.sync_copy(acc.at[pl.ds(sid * _ZS, _ZS)],
                        o_ref.at[cid, pl.ds(sid * _ZS, _ZS)])

    return k(xwa, xwb, dinv16, row_t, col_t, zpad)


def _tc_matmul(x, W):
    def body(x_ref, w_ref, a_ref, b_ref):
        xw = jnp.dot(x_ref[...], w_ref[...], preferred_element_type=jnp.float32)
        a_ref[...] = xw[:, :_HC]
        b_ref[...] = xw[:, _HC:]

    return pl.pallas_call(
        body,
        out_shape=(jax.ShapeDtypeStruct((_N, _HC), jnp.float32),
                   jax.ShapeDtypeStruct((_N, _HC), jnp.float32)),
    )(x, W)


def _tc_final(t, dinv, b2):
    def body(t_ref, dv_ref, b_ref, o_ref):
        dv = dv_ref[...]
        o_ref[...] = (jnp.concatenate([t_ref[0] * dv, t_ref[1] * dv], axis=1)
                      + b_ref[...])

    return pl.pallas_call(
        body,
        out_shape=jax.ShapeDtypeStruct((_N, _C), jnp.float32),
    )(t, dinv, b2)


def kernel(x, edge_index, W, b):
    row = edge_index[0]
    col = edge_index[1]
    pad = _EPAD - _E
    rowp = jnp.concatenate([row, jnp.zeros((pad,), row.dtype)])
    colp = jnp.concatenate([col, jnp.full((pad,), _N, col.dtype)])
    row_t16 = rowp.reshape(_NS, _CHH, _CH)
    col_t16 = colp.reshape(_NS, _CHH, _CH)
    z16 = jnp.zeros((_ZS, 16), jnp.float32)
    ones16 = jnp.ones((_CH, 16), jnp.float32)
    zpad = jnp.zeros((_NPAD - _N, _HC), jnp.float32)

    dinv16 = _sc_degree_dinv(col_t16, z16, ones16)
    xwa, xwb = _tc_matmul(x, W)
    t = _sc_hops(xwa, xwb, dinv16, row_t16, col_t16, zpad)
    return _tc_final(t[:, :_N], dinv16[:_N, :1], b.reshape(1, _C))


# speedup vs baseline: 46.6825x; 1.0860x over previous
import functools

import jax
import jax.numpy as jnp
from jax import lax
from jax.experimental import pallas as pl
from jax.experimental.pallas import tpu as pltpu
from jax.experimental.pallas import tpu_sc as plsc

_N = 10000
_D = 128
_C = 64
_HC = _C // 2
_E = 320000
_NC = 2
_NS = 16
_NW = _NC * _NS
_CH = 128
_CHD = 80
_CHH = 160
_EPAD = _NW * _CHD * _CH
_NPAD = 10112
_ZS = _NPAD // _NS
_NB = 4
_PD = 2
_LASTY = 15 * _ZS
_LASTN = _N - _LASTY


def _vmesh():
    return plsc.VectorSubcoreMesh(core_axis_name="c", subcore_axis_name="s")


def _newton_rsqrt(v):
    i = lax.bitcast_convert_type(v, jnp.int32)
    i = 0x5F3759DF - lax.shift_right_logical(i, 1)
    g = lax.bitcast_convert_type(i, jnp.float32)
    h = v * 0.5
    for _ in range(3):
        g = g * (1.5 - h * g * g)
    return g


def _sc_degree(col_t, z16, ones16):
    @functools.partial(
        pl.kernel,
        out_type=jax.ShapeDtypeStruct((_NC, _NPAD, 16), jnp.float32),
        mesh=_vmesh(),
        compiler_params=pltpu.CompilerParams(use_tc_tiling_on_sc=False),
        scratch_types=[
            pltpu.VMEM((_CHD, _CH), jnp.int32),
            pltpu.VMEM((_CH, 16), jnp.float32),
            pltpu.VMEM_SHARED((_NPAD, 16), jnp.float32),
        ],
    )
    def k(col_ref, z_ref, ones_ref, out_ref, col_v, ones_v, acc):
        cid = lax.axis_index("c")
        sid = lax.axis_index("s")
        wid = sid * _NC + cid
        pltpu.sync_copy(z_ref, acc.at[pl.ds(sid * _ZS, _ZS)])
        pltpu.sync_copy(col_ref.at[wid], col_v)
        pltpu.sync_copy(ones_ref, ones_v)
        plsc.subcore_barrier()

        @pl.loop(0, _CHD)
        def _(j):
            pltpu.sync_copy(ones_v, acc.at[col_v.at[j]], add=True)

        plsc.subcore_barrier()
        pltpu.sync_copy(acc.at[pl.ds(sid * _ZS, _ZS)],
                        out_ref.at[cid, pl.ds(sid * _ZS, _ZS)])

    return k(col_t, z16, ones16)


def _sc_hops(xwa, xwb, degp, row_t, col_t, zpad, b4):
    @functools.partial(
        pl.kernel,
        out_type=jax.ShapeDtypeStruct((_N, _C), jnp.float32),
        mesh=_vmesh(),
        compiler_params=pltpu.CompilerParams(use_tc_tiling_on_sc=False),
        scratch_types=[
            pltpu.VMEM((_CHH, _CH), jnp.int32),
            pltpu.VMEM((_CHH, _CH), jnp.int32),
            [pltpu.VMEM((_CH, _HC), jnp.float32) for _ in range(_NB)],
            pltpu.VMEM((_ZS, _HC), jnp.float32),
            pltpu.VMEM((_ZS, 16), jnp.float32),
            pltpu.VMEM((2, 16), jnp.float32),
            pltpu.SemaphoreType.DMA((_NB,)),
            pltpu.SemaphoreType.DMA((_NB,)),
            pltpu.VMEM_SHARED((_NPAD, _HC), jnp.float32),
            pltpu.VMEM_SHARED((_N, _HC), jnp.float32),
        ],
    )
    def k(xwa_ref, xwb_ref, dp_ref, row_ref, col_ref, zpad_ref, b_ref, o_ref,
          row_v, col_v, gb, ybuf, dv, bbuf, semg, sems, acc, ysh):
        cid = lax.axis_index("c")
        sid = lax.axis_index("s")
        pltpu.sync_copy(row_ref.at[sid], row_v)
        pltpu.sync_copy(col_ref.at[sid], col_v)
        pltpu.sync_copy(b_ref.at[pl.ds(cid * 2, 2)], bbuf)

        sl = pl.ds(sid * _ZS, _ZS)
        pltpu.sync_copy(dp_ref.at[0, sl], dv)
        pltpu.sync_copy(dp_ref.at[1, sl], ybuf.at[:, pl.ds(0, 16)])

        @pl.loop(0, _ZS)
        def _(r):
            dv[r] = _newton_rsqrt(dv[r] + ybuf[r, pl.ds(0, 16)] + 1.0)

        def scale_rows(n):
            @pl.loop(0, n)
            def _(r):
                s = dv[r]
                for q in (0, 16):
                    ybuf[r, pl.ds(q, 16)] = ybuf[r, pl.ds(q, 16)] * s

        def stage(xw_ref):
            @pl.when(sid < _NS - 1)
            def _():
                pltpu.sync_copy(xw_ref.at[sl], ybuf)
                scale_rows(_ZS)
                pltpu.sync_copy(ybuf, ysh.at[sl])
                pltpu.sync_copy(ybuf, acc.at[sl])

            @pl.when(sid == _NS - 1)
            def _():
                ll = pl.ds(_LASTY, _LASTN)
                bl = pl.ds(0, _LASTN)
                pltpu.sync_copy(xw_ref.at[ll], ybuf.at[bl])
                scale_rows(_LASTN)
                pltpu.sync_copy(ybuf.at[bl], ysh.at[ll])
                pltpu.sync_copy(ybuf.at[bl], acc.at[ll])
                pltpu.sync_copy(zpad_ref, acc.at[pl.ds(_N, _NPAD - _N)])

        @pl.when(cid == 0)
        def _():
            stage(xwa_ref)

        @pl.when(cid == 1)
        def _():
            stage(xwb_ref)

        plsc.subcore_barrier()

        def gather(c, p):
            pltpu.async_copy(ysh.at[row_v.at[c]], gb[p], semg.at[p])

        def wait_gather(c, p):
            pltpu.make_async_copy(ysh.at[row_v.at[c]], gb[p],
                                  semg.at[p]).wait()

        def scatter(c, p):
            pltpu.async_copy(gb[p], acc.at[col_v.at[c]], sems.at[p], add=True)

        def wait_scatter(c, p):
            pltpu.make_async_copy(gb[p], acc.at[col_v.at[c]],
                                  sems.at[p]).wait()

        def hop_loop():
            for p in range(_PD):
                gather(p, p)

            @pl.loop(0, _CHH // _NB)
            def _(t):
                j0 = t * _NB
                for p in range(_NB):
                    j = j0 + p
                    pn = (p + _PD) % _NB

                    @pl.when(j + _PD >= _NB)
                    def _():
                        wait_scatter(j + _PD - _NB, pn)

                    @pl.when(j + _PD < _CHH)
                    def _():
                        gather(j + _PD, pn)

                    wait_gather(j, p)
                    scatter(j, p)

            for c in range(_CHH - _NB, _CHH):
                if c + _NB - _PD > _CHH - 1:
                    wait_scatter(c, c % _NB)

            plsc.subcore_barrier()

        hop_loop()

        pltpu.sync_copy(acc.at[sl], ybuf)

        @pl.loop(0, _ZS)
        def _(r):
            s = dv[r]
            s2 = s * s
            for q in (0, 16):
                ybuf[r, pl.ds(q, 16)] = ybuf[r, pl.ds(q, 16)] * s2

        pltpu.sync_copy(ybuf, acc.at[sl])

        @pl.when(sid < _NS - 1)
        def _():
            pltpu.sync_copy(ybuf, ysh.at[sl])

        @pl.when(sid == _NS - 1)
        def _():
            pltpu.sync_copy(ybuf.at[pl.ds(0, _LASTN)],
                            ysh.at[pl.ds(_LASTY, _LASTN)])

        plsc.subcore_barrier()

        hop_loop()

        pltpu.sync_copy(acc.at[sl], ybuf)

        @pl.loop(0, _ZS)
        def _(r):
            s = dv[r]
            for qi in (0, 1):
                q = qi * 16
                ybuf[r, pl.ds(q, 16)] = (ybuf[r, pl.ds(q, 16)] * s
                                         + bbuf[qi])

        cslice = pl.ds(cid * _HC, _HC)

        @pl.when(sid < _NS - 1)
        def _():
            pltpu.sync_copy(ybuf, o_ref.at[sl, cslice])

        @pl.when(sid == _NS - 1)
        def _():
            pltpu.sync_copy(ybuf.at[pl.ds(0, _LASTN)],
                            o_ref.at[pl.ds(_LASTY, _LASTN), cslice])

    return k(xwa, xwb, degp, row_t, col_t, zpad, b4)


def _tc_matmul(x, W):
    def body(x_ref, w_ref, a_ref, b_ref):
        xw = jnp.dot(x_ref[...], w_ref[...], preferred_element_type=jnp.float32)
        a_ref[...] = xw[:, :_HC]
        b_ref[...] = xw[:, _HC:]

    return pl.pallas_call(
        body,
        out_shape=(jax.ShapeDtypeStruct((_N, _HC), jnp.float32),
                   jax.ShapeDtypeStruct((_N, _HC), jnp.float32)),
    )(x, W)


def kernel(x, edge_index, W, b):
    row = edge_index[0]
    col = edge_index[1]
    pad = _EPAD - _E
    rowp = jnp.concatenate([row, jnp.zeros((pad,), row.dtype)])
    colp = jnp.concatenate([col, jnp.full((pad,), _N, col.dtype)])
    col_t32 = colp.reshape(_NW, _CHD, _CH)
    row_t16 = rowp.reshape(_NS, _CHH, _CH)
    col_t16 = colp.reshape(_NS, _CHH, _CH)
    z16 = jnp.zeros((_ZS, 16), jnp.float32)
    ones16 = jnp.ones((_CH, 16), jnp.float32)
    zpad = jnp.zeros((_NPAD - _N, _HC), jnp.float32)

    degp = _sc_degree(col_t32, z16, ones16)
    xwa, xwb = _tc_matmul(x, W)
    return _sc_hops(xwa, xwb, degp, row_t16, col_t16, zpad,
                    b.astype(jnp.float32).reshape(4, 16))
